# baseline (speedup 1.0000x reference)
import jax
import jax.numpy as jnp
from jax.experimental import pallas as pl
from jax.experimental.pallas import tpu as pltpu
from jax.experimental.pallas import tpu_sc as plsc

_ROWS = 8
_LANES = 16


def kernel(inputs, pos_table):
    batch, seq, emb = inputs.shape
    x = inputs.reshape(batch * seq, emb)
    nseq = seq // _ROWS
    mesh = plsc.VectorSubcoreMesh(core_axis_name="c", subcore_axis_name="s")

    @pl.kernel(
        out_type=jax.ShapeDtypeStruct((batch * seq, emb), inputs.dtype),
        mesh=mesh,
    )
    def sc_kernel(x_hbm, p_hbm, o_hbm):
        def body(x_vmem, p_vmem, o_vmem):
            @pl.loop(0, _ROWS)
            def _(r):
                @pl.loop(0, emb, step=_LANES)
                def _(c):
                    slc = (pl.ds(r, 1), pl.ds(c, _LANES))
                    o_vmem.at[*slc][...] = (
                        x_vmem.at[*slc][...] + p_vmem.at[*slc][...]
                    )

        pltpu.emit_pipeline(
            body,
            grid=(nseq, batch),
            in_specs=[
                pl.BlockSpec((_ROWS, emb), index_map=lambda i, j: (j * nseq + i, 0)),
                pl.BlockSpec((_ROWS, emb), index_map=lambda i, j: (i, 0)),
            ],
            out_specs=[
                pl.BlockSpec((_ROWS, emb), index_map=lambda i, j: (j * nseq + i, 0)),
            ],
            core_axis_name=("c", "s"),
            dimension_semantics=(pltpu.PARALLEL, pltpu.ARBITRARY),
        )(x_hbm, p_hbm, o_hbm)

    out = sc_kernel(x, pos_table)
    return out.reshape(batch, seq, emb)


# speedup vs baseline: 1.0113x; 1.0113x over previous
import jax
import jax.numpy as jnp
from jax.experimental import pallas as pl
from jax.experimental.pallas import tpu as pltpu
from jax.experimental.pallas import tpu_sc as plsc

_ROWS = 8
_LANES = 16


def kernel(inputs, pos_table):
    batch, seq, emb = inputs.shape
    x = inputs.reshape(batch * seq, emb)
    nseq = seq // _ROWS
    mesh = plsc.VectorSubcoreMesh(core_axis_name="c", subcore_axis_name="s")

    @pl.kernel(
        out_type=jax.ShapeDtypeStruct((batch * seq, emb), inputs.dtype),
        mesh=mesh,
    )
    def sc_kernel(x_hbm, p_hbm, o_hbm):
        def body(x_vmem, p_vmem, o_vmem):
            @pl.loop(0, _ROWS)
            def _(r):
                for c in range(0, emb, _LANES):
                    slc = (pl.ds(r, 1), pl.ds(c, _LANES))
                    o_vmem.at[*slc][...] = (
                        x_vmem.at[*slc][...] + p_vmem.at[*slc][...]
                    )

        pltpu.emit_pipeline(
            body,
            grid=(nseq, batch),
            in_specs=[
                pl.BlockSpec((_ROWS, emb), index_map=lambda i, j: (j * nseq + i, 0)),
                pl.BlockSpec((_ROWS, emb), index_map=lambda i, j: (i, 0)),
            ],
            out_specs=[
                pl.BlockSpec((_ROWS, emb), index_map=lambda i, j: (j * nseq + i, 0)),
            ],
            core_axis_name=("c", "s"),
            dimension_semantics=(pltpu.PARALLEL, pltpu.ARBITRARY),
        )(x_hbm, p_hbm, o_hbm)

    out = sc_kernel(x, pos_table)
    return out.reshape(batch, seq, emb)
